# baseline (speedup 1.0000x reference)
import functools

import jax
import jax.numpy as jnp
from jax import lax
from jax.experimental import pallas as pl
from jax.experimental.pallas import tpu as pltpu
from jax.experimental.pallas import tpu_sc as plsc

VOCAB = 100000
EMB = 256
IMG_DIM = 4096
HID = 256
B = 1024
BH = B // 2
L = 50

_N_IDX = B * L
_GATHER_WINDOW = 128

_V_TILE = 2048
_N_VTILES = (VOCAB + _V_TILE - 1) // _V_TILE


def _sc_gather(table, idx_flat):
    idx2d = idx_flat.reshape(1, _N_IDX)
    mesh = plsc.VectorSubcoreMesh(core_axis_name="c", subcore_axis_name="s")

    @functools.partial(
        pl.kernel,
        out_type=jax.ShapeDtypeStruct((_N_IDX, EMB), jnp.float32),
        mesh=mesh,
    )
    def gather_kernel(table_hbm, idx_hbm, out_hbm):
        def body(i_vmem, o_vmem):
            pltpu.sync_copy(table_hbm.at[i_vmem.at[0]], o_vmem)

        pltpu.emit_pipeline(
            body,
            grid=(_N_IDX // _GATHER_WINDOW,),
            in_specs=[pl.BlockSpec((1, _GATHER_WINDOW), lambda i: (0, i))],
            out_specs=[pl.BlockSpec((_GATHER_WINDOW, EMB), lambda i: (i, 0))],
            core_axis_name=("c", "s"),
            dimension_semantics=(pltpu.PARALLEL,),
        )(idx_hbm, out_hbm)

    return gather_kernel(table, idx2d)


def _dot_t(a, b):
    return lax.dot_general(
        a.astype(jnp.bfloat16),
        b.astype(jnp.bfloat16),
        (((1,), (1,)), ((), ())),
        preferred_element_type=jnp.float32,
    )


def _lstm_step(x, h_ref, c_ref, wih_ref, whh_ref, bg_ref):
    gates = (
        _dot_t(x, wih_ref[...]) + _dot_t(h_ref[...], whh_ref[...]) + bg_ref[...]
    ).astype(jnp.bfloat16)
    i_g = jax.nn.sigmoid(gates[:, 0:HID])
    f_g = jax.nn.sigmoid(gates[:, HID : 2 * HID])
    g_g = jnp.tanh(gates[:, 2 * HID : 3 * HID])
    o_g = jax.nn.sigmoid(gates[:, 3 * HID : 4 * HID])
    c_new = (f_g * c_ref[...].astype(jnp.bfloat16) + i_g * g_g).astype(jnp.float32)
    h_new = o_g.astype(jnp.float32) * jnp.tanh(c_new)
    c_ref[...] = c_new
    h_ref[...] = h_new.astype(jnp.bfloat16)
    return h_new


_T_PER_STEP = 2


def _lstm_a_body(
    embd_ref, image_ref, wfc_ref, bfc_ref, wih_ref, whh_ref, bg_ref,
    wd1_ref, bd1_ref, out_ref, h_ref, c_ref, img_ref,
):
    t = pl.program_id(0)

    @pl.when(t == 0)
    def _init():
        img_ref[...] = jnp.maximum(
            _dot_t(image_ref[...], wfc_ref[...]) + bfc_ref[...], 0.0
        )
        h_ref[...] = jnp.zeros_like(h_ref)
        c_ref[...] = jnp.zeros_like(c_ref)

    for j in range(_T_PER_STEP):
        h_new = _lstm_step(embd_ref[j], h_ref, c_ref, wih_ref, whh_ref, bg_ref)

    @pl.when(t == L // _T_PER_STEP - 1)
    def _finish():
        df = img_ref[...] + h_new
        out_ref[...] = jnp.maximum(_dot_t(df, wd1_ref[...]) + bd1_ref[...], 0.0)


def _lstm_a(embd_tm, image, W_fc, b_fc, W_ih, W_hh, b_gates, W_d1, b_d1):
    return pl.pallas_call(
        _lstm_a_body,
        grid=(L // _T_PER_STEP,),
        in_specs=[
            pl.BlockSpec((_T_PER_STEP, BH, EMB), lambda t: (t, 0, 0)),
            pl.BlockSpec((BH, IMG_DIM), lambda t: (0, 0)),
            pl.BlockSpec((EMB, IMG_DIM), lambda t: (0, 0)),
            pl.BlockSpec((1, EMB), lambda t: (0, 0)),
            pl.BlockSpec((4 * HID, EMB), lambda t: (0, 0)),
            pl.BlockSpec((4 * HID, HID), lambda t: (0, 0)),
            pl.BlockSpec((1, 4 * HID), lambda t: (0, 0)),
            pl.BlockSpec((EMB, EMB), lambda t: (0, 0)),
            pl.BlockSpec((1, EMB), lambda t: (0, 0)),
        ],
        out_specs=pl.BlockSpec((BH, EMB), lambda t: (0, 0)),
        out_shape=jax.ShapeDtypeStruct((BH, EMB), jnp.float32),
        scratch_shapes=[
            pltpu.VMEM((BH, HID), jnp.bfloat16),
            pltpu.VMEM((BH, HID), jnp.float32),
            pltpu.VMEM((BH, EMB), jnp.float32),
        ],
        compiler_params=pltpu.CompilerParams(
            dimension_semantics=("arbitrary",),
        ),
    )(embd_tm, image, W_fc, b_fc, W_ih, W_hh, b_gates, W_d1, b_d1)


def _mega_body(
    embd_ref, image_ref, wfc_ref, bfc_ref, wih_ref, whh_ref, bg_ref,
    wd1_ref, bd1_ref, h1a_ref, wd2_ref, bd2_ref,
    out_ref, h1b_ref, h_ref, c_ref, img_ref,
):
    v = pl.program_id(0)

    @pl.when(v == 0)
    def _init():
        img_ref[...] = jnp.maximum(
            _dot_t(image_ref[...], wfc_ref[...]) + bfc_ref[...], 0.0
        )
        h_ref[...] = jnp.zeros_like(h_ref)
        c_ref[...] = jnp.zeros_like(c_ref)

    out_ref[...] = _dot_t(h1a_ref[...], wd2_ref[...]) + bd2_ref[...]

    h_new = _lstm_step(embd_ref[0], h_ref, c_ref, wih_ref, whh_ref, bg_ref)

    @pl.when(v == L - 1)
    def _finish():
        df = img_ref[...] + h_new
        h1b_ref[...] = jnp.maximum(_dot_t(df, wd1_ref[...]) + bd1_ref[...], 0.0)


def _mega_a(embd_tm, image, W_fc, b_fc, W_ih, W_hh, b_gates, W_d1, b_d1, h1a, W_d2, b_d2):
    vt = lambda v: jnp.minimum(v, _N_VTILES - 1)
    return pl.pallas_call(
        _mega_body,
        grid=(L,),
        in_specs=[
            pl.BlockSpec((1, BH, EMB), lambda v: (v, 1, 0)),
            pl.BlockSpec((BH, IMG_DIM), lambda v: (1, 0)),
            pl.BlockSpec((EMB, IMG_DIM), lambda v: (0, 0)),
            pl.BlockSpec((1, EMB), lambda v: (0, 0)),
            pl.BlockSpec((4 * HID, EMB), lambda v: (0, 0)),
            pl.BlockSpec((4 * HID, HID), lambda v: (0, 0)),
            pl.BlockSpec((1, 4 * HID), lambda v: (0, 0)),
            pl.BlockSpec((EMB, EMB), lambda v: (0, 0)),
            pl.BlockSpec((1, EMB), lambda v: (0, 0)),
            pl.BlockSpec((BH, EMB), lambda v: (0, 0)),
            pl.BlockSpec((_V_TILE, EMB), lambda v: (vt(v), 0)),
            pl.BlockSpec((1, _V_TILE), lambda v: (0, vt(v))),
        ],
        out_specs=[
            pl.BlockSpec((BH, _V_TILE), lambda v: (0, vt(v))),
            pl.BlockSpec((BH, EMB), lambda v: (0, 0)),
        ],
        out_shape=[
            jax.ShapeDtypeStruct((B, VOCAB), jnp.float32),
            jax.ShapeDtypeStruct((BH, EMB), jnp.float32),
        ],
        scratch_shapes=[
            pltpu.VMEM((BH, HID), jnp.bfloat16),
            pltpu.VMEM((BH, HID), jnp.float32),
            pltpu.VMEM((BH, EMB), jnp.float32),
        ],
        compiler_params=pltpu.CompilerParams(
            dimension_semantics=("arbitrary",),
        ),
    )(embd_tm, image, W_fc, b_fc, W_ih, W_hh, b_gates, W_d1, b_d1, h1a, W_d2, b_d2)


def _proj_b_body(h1b_ref, wd2_ref, bd2_ref, outa_ref, out_ref):
    out_ref[...] = _dot_t(h1b_ref[...], wd2_ref[...]) + bd2_ref[...]


def _proj_b(h1b, W_d2, b_d2, out_a):
    return pl.pallas_call(
        _proj_b_body,
        grid=(_N_VTILES,),
        in_specs=[
            pl.BlockSpec((BH, EMB), lambda v: (0, 0)),
            pl.BlockSpec((_V_TILE, EMB), lambda v: (v, 0)),
            pl.BlockSpec((1, _V_TILE), lambda v: (0, v)),
            pl.BlockSpec(memory_space=pltpu.HBM),
        ],
        out_specs=pl.BlockSpec((BH, _V_TILE), lambda v: (1, v)),
        out_shape=jax.ShapeDtypeStruct((B, VOCAB), jnp.float32),
        input_output_aliases={3: 0},
        compiler_params=pltpu.CompilerParams(
            dimension_semantics=("arbitrary",),
        ),
    )(h1b, W_d2, b_d2, out_a)


def kernel(image, caption, W_fc, b_fc, emb, W_ih, W_hh, b_ih, b_hh, W_d1, b_d1, W_d2, b_d2):
    idx_flat = caption.astype(jnp.int32).T.reshape(-1)
    embd = _sc_gather(emb, idx_flat)
    embd_tm = embd.reshape(L, B, EMB)
    image_b = image.astype(jnp.bfloat16)
    W_fc_b = W_fc.astype(jnp.bfloat16)
    W_ih_b = W_ih.astype(jnp.bfloat16)
    W_hh_b = W_hh.astype(jnp.bfloat16)
    W_d1_b = W_d1.astype(jnp.bfloat16)
    bfc = b_fc.reshape(1, EMB)
    bg = (b_ih + b_hh).reshape(1, 4 * HID)
    bd1 = b_d1.reshape(1, EMB)
    bd2 = b_d2.reshape(1, VOCAB)
    h1a = _lstm_a(embd_tm, image_b, W_fc_b, bfc, W_ih_b, W_hh_b, bg, W_d1_b, bd1)
    out_a, h1b = _mega_a(
        embd_tm, image_b, W_fc_b, bfc, W_ih_b, W_hh_b, bg, W_d1_b, bd1,
        h1a, W_d2, bd2,
    )
    return _proj_b(h1b, W_d2, bd2, out_a)


# speedup vs baseline: 1.0426x; 1.0426x over previous
import functools

import jax
import jax.numpy as jnp
from jax import lax
from jax.experimental import pallas as pl
from jax.experimental.pallas import tpu as pltpu
from jax.experimental.pallas import tpu_sc as plsc

VOCAB = 100000
EMB = 256
IMG_DIM = 4096
HID = 256
B = 1024
L = 50

_SC_CORES = 2
_SC_SUBCORES = 16
_NW = _SC_CORES * _SC_SUBCORES

_N_IDX = B * L
_T_PER_STEP = 2
_GATHER_WINDOW = 128


def _sc_gather(table, idx_flat, n):
    idx2d = idx_flat.reshape(1, n)
    mesh = plsc.VectorSubcoreMesh(core_axis_name="c", subcore_axis_name="s")

    @functools.partial(
        pl.kernel,
        out_type=jax.ShapeDtypeStruct((n, EMB), jnp.float32),
        mesh=mesh,
    )
    def gather_kernel(table_hbm, idx_hbm, out_hbm):
        def body(i_vmem, o_vmem):
            pltpu.sync_copy(table_hbm.at[i_vmem.at[0]], o_vmem)

        pltpu.emit_pipeline(
            body,
            grid=(n // _GATHER_WINDOW,),
            in_specs=[pl.BlockSpec((1, _GATHER_WINDOW), lambda i: (0, i))],
            out_specs=[pl.BlockSpec((_GATHER_WINDOW, EMB), lambda i: (i, 0))],
            core_axis_name=("c", "s"),
            dimension_semantics=(pltpu.PARALLEL,),
        )(idx_hbm, out_hbm)

    return gather_kernel(table, idx2d)


def _dot_t(a, b):
    return lax.dot_general(
        a.astype(jnp.bfloat16),
        b.astype(jnp.bfloat16),
        (((1,), (1,)), ((), ())),
        preferred_element_type=jnp.float32,
    )


_N_CHUNKS = 5
_CH_STEPS = L // _N_CHUNKS
_CH_GRID = _CH_STEPS // _T_PER_STEP


def _lstm_steps(embd_ref, h_ref, c_ref, wih_ref, whh_ref, bg_ref):
    h_new = None
    for j in range(_T_PER_STEP):
        x = embd_ref[j]
        gates = (
            _dot_t(x, wih_ref[...]) + _dot_t(h_ref[...], whh_ref[...]) + bg_ref[...]
        ).astype(jnp.bfloat16)
        i_g = jax.nn.sigmoid(gates[:, 0:HID])
        f_g = jax.nn.sigmoid(gates[:, HID : 2 * HID])
        g_g = jnp.tanh(gates[:, 2 * HID : 3 * HID])
        o_g = jax.nn.sigmoid(gates[:, 3 * HID : 4 * HID])
        c_new = (f_g * c_ref[...].astype(jnp.bfloat16) + i_g * g_g).astype(jnp.float32)
        h_new = o_g.astype(jnp.float32) * jnp.tanh(c_new)
        c_ref[...] = c_new
        h_ref[...] = h_new.astype(jnp.bfloat16)
    return h_new


def _chunk_body(
    embd_ref, hin_ref, cin_ref, wih_ref, whh_ref, bg_ref,
    hout_ref, cout_ref, h_ref, c_ref,
):
    t = pl.program_id(0)

    @pl.when(t == 0)
    def _init():
        h_ref[...] = hin_ref[...]
        c_ref[...] = cin_ref[...]

    _lstm_steps(embd_ref, h_ref, c_ref, wih_ref, whh_ref, bg_ref)

    @pl.when(t == _CH_GRID - 1)
    def _finish():
        hout_ref[...] = h_ref[...]
        cout_ref[...] = c_ref[...]


def _lstm_chunk(embd_c, h, c, W_ih, W_hh, b_gates):
    return pl.pallas_call(
        _chunk_body,
        grid=(_CH_GRID,),
        in_specs=[
            pl.BlockSpec((_T_PER_STEP, B, EMB), lambda t: (t, 0, 0)),
            pl.BlockSpec((B, HID), lambda t: (0, 0)),
            pl.BlockSpec((B, HID), lambda t: (0, 0)),
            pl.BlockSpec((4 * HID, EMB), lambda t: (0, 0)),
            pl.BlockSpec((4 * HID, HID), lambda t: (0, 0)),
            pl.BlockSpec((1, 4 * HID), lambda t: (0, 0)),
        ],
        out_specs=[
            pl.BlockSpec((B, HID), lambda t: (0, 0)),
            pl.BlockSpec((B, HID), lambda t: (0, 0)),
        ],
        out_shape=[
            jax.ShapeDtypeStruct((B, HID), jnp.bfloat16),
            jax.ShapeDtypeStruct((B, HID), jnp.float32),
        ],
        scratch_shapes=[
            pltpu.VMEM((B, HID), jnp.bfloat16),
            pltpu.VMEM((B, HID), jnp.float32),
        ],
        compiler_params=pltpu.CompilerParams(
            dimension_semantics=("arbitrary",),
        ),
    )(embd_c, h, c, W_ih, W_hh, b_gates)


def _last_body(
    embd_ref, hin_ref, cin_ref, wih_ref, whh_ref, bg_ref,
    image_ref, wfc_ref, bfc_ref, wd1_ref, bd1_ref,
    out_ref, h_ref, c_ref, img_ref,
):
    t = pl.program_id(0)

    @pl.when(t == 0)
    def _init():
        h_ref[...] = hin_ref[...]
        c_ref[...] = cin_ref[...]
        img_ref[...] = jnp.maximum(
            _dot_t(image_ref[...], wfc_ref[...]) + bfc_ref[...], 0.0
        )

    h_new = _lstm_steps(embd_ref, h_ref, c_ref, wih_ref, whh_ref, bg_ref)

    @pl.when(t == _CH_GRID - 1)
    def _finish():
        df = img_ref[...] + h_new
        out_ref[...] = jnp.maximum(_dot_t(df, wd1_ref[...]) + bd1_ref[...], 0.0)


def _lstm_last(embd_c, h, c, W_ih, W_hh, b_gates, image, W_fc, b_fc, W_d1, b_d1):
    return pl.pallas_call(
        _last_body,
        grid=(_CH_GRID,),
        in_specs=[
            pl.BlockSpec((_T_PER_STEP, B, EMB), lambda t: (t, 0, 0)),
            pl.BlockSpec((B, HID), lambda t: (0, 0)),
            pl.BlockSpec((B, HID), lambda t: (0, 0)),
            pl.BlockSpec((4 * HID, EMB), lambda t: (0, 0)),
            pl.BlockSpec((4 * HID, HID), lambda t: (0, 0)),
            pl.BlockSpec((1, 4 * HID), lambda t: (0, 0)),
            pl.BlockSpec((B, IMG_DIM), lambda t: (0, 0)),
            pl.BlockSpec((EMB, IMG_DIM), lambda t: (0, 0)),
            pl.BlockSpec((1, EMB), lambda t: (0, 0)),
            pl.BlockSpec((EMB, EMB), lambda t: (0, 0)),
            pl.BlockSpec((1, EMB), lambda t: (0, 0)),
        ],
        out_specs=pl.BlockSpec((B, EMB), lambda t: (0, 0)),
        out_shape=jax.ShapeDtypeStruct((B, EMB), jnp.float32),
        scratch_shapes=[
            pltpu.VMEM((B, HID), jnp.bfloat16),
            pltpu.VMEM((B, HID), jnp.float32),
            pltpu.VMEM((B, EMB), jnp.float32),
        ],
        compiler_params=pltpu.CompilerParams(
            dimension_semantics=("arbitrary",),
        ),
    )(embd_c, h, c, W_ih, W_hh, b_gates, image, W_fc, b_fc, W_d1, b_d1)


_V_TILE = 2048
_N_VTILES = (VOCAB + _V_TILE - 1) // _V_TILE


def _proj_body(h1_ref, wd2_ref, bd2_ref, out_ref):
    out_ref[...] = _dot_t(h1_ref[...], wd2_ref[...]) + bd2_ref[...]


def _vocab_proj(h1, W_d2, b_d2):
    return pl.pallas_call(
        _proj_body,
        grid=(_N_VTILES,),
        in_specs=[
            pl.BlockSpec((B, EMB), lambda v: (0, 0)),
            pl.BlockSpec((_V_TILE, EMB), lambda v: (v, 0)),
            pl.BlockSpec((1, _V_TILE), lambda v: (0, v)),
        ],
        out_specs=pl.BlockSpec((B, _V_TILE), lambda v: (0, v)),
        out_shape=jax.ShapeDtypeStruct((B, VOCAB), jnp.float32),
        compiler_params=pltpu.CompilerParams(
            dimension_semantics=("arbitrary",),
        ),
    )(h1, W_d2, b_d2)


def kernel(image, caption, W_fc, b_fc, emb, W_ih, W_hh, b_ih, b_hh, W_d1, b_d1, W_d2, b_d2):
    idx_flat = caption.astype(jnp.int32).T.reshape(-1)
    n_ch = B * _CH_STEPS
    W_ih_b = W_ih.astype(jnp.bfloat16)
    W_hh_b = W_hh.astype(jnp.bfloat16)
    bg = (b_ih + b_hh).reshape(1, 4 * HID)
    h = jnp.zeros((B, HID), jnp.bfloat16)
    c = jnp.zeros((B, HID), jnp.float32)
    for ci in range(_N_CHUNKS):
        embd_c = _sc_gather(emb, idx_flat[ci * n_ch : (ci + 1) * n_ch], n_ch)
        embd_c = embd_c.reshape(_CH_STEPS, B, EMB)
        if ci < _N_CHUNKS - 1:
            h, c = _lstm_chunk(embd_c, h, c, W_ih_b, W_hh_b, bg)
        else:
            h1 = _lstm_last(
                embd_c, h, c, W_ih_b, W_hh_b, bg,
                image.astype(jnp.bfloat16),
                W_fc.astype(jnp.bfloat16),
                b_fc.reshape(1, EMB),
                W_d1.astype(jnp.bfloat16),
                b_d1.reshape(1, EMB),
            )
    return _vocab_proj(h1, W_d2, b_d2.reshape(1, VOCAB))


# speedup vs baseline: 1.0483x; 1.0055x over previous
import functools

import jax
import jax.numpy as jnp
from jax import lax
from jax.experimental import pallas as pl
from jax.experimental.pallas import tpu as pltpu
from jax.experimental.pallas import tpu_sc as plsc

VOCAB = 100000
EMB = 256
IMG_DIM = 4096
HID = 256
B = 1024
L = 50

_SC_CORES = 2
_SC_SUBCORES = 16
_NW = _SC_CORES * _SC_SUBCORES

_N_IDX = B * L
_T_PER_STEP = 2
_GATHER_WINDOW = 128


def _sc_gather(table, idx_flat):
    idx2d = idx_flat.reshape(1, _N_IDX)
    mesh = plsc.VectorSubcoreMesh(core_axis_name="c", subcore_axis_name="s")

    @functools.partial(
        pl.kernel,
        out_type=jax.ShapeDtypeStruct((_N_IDX, EMB), jnp.float32),
        mesh=mesh,
    )
    def gather_kernel(table_hbm, idx_hbm, out_hbm):
        def body(i_vmem, o_vmem):
            pltpu.sync_copy(table_hbm.at[i_vmem.at[0]], o_vmem)

        pltpu.emit_pipeline(
            body,
            grid=(_N_IDX // _GATHER_WINDOW,),
            in_specs=[pl.BlockSpec((1, _GATHER_WINDOW), lambda i: (0, i))],
            out_specs=[pl.BlockSpec((_GATHER_WINDOW, EMB), lambda i: (i, 0))],
            core_axis_name=("c", "s"),
            dimension_semantics=(pltpu.PARALLEL,),
        )(idx_hbm, out_hbm)

    return gather_kernel(table, idx2d)


def _dot_t(a, b):
    return lax.dot_general(
        a.astype(jnp.bfloat16),
        b.astype(jnp.bfloat16),
        (((1,), (1,)), ((), ())),
        preferred_element_type=jnp.float32,
    )


def _lstm_body(
    embd_ref, image_ref, wfc_ref, bfc_ref, wih_ref, whh_ref, bg_ref,
    wd1_ref, bd1_ref, out_ref, h_ref, c_ref, img_ref,
):
    t = pl.program_id(0)

    @pl.when(t == 0)
    def _init():
        img_ref[...] = jnp.maximum(
            _dot_t(image_ref[...], wfc_ref[...]) + bfc_ref[...], 0.0
        )
        h_ref[...] = jnp.zeros_like(h_ref)
        c_ref[...] = jnp.zeros_like(c_ref)

    for j in range(_T_PER_STEP):
        x = embd_ref[j]
        gates = (
            _dot_t(x, wih_ref[...]) + _dot_t(h_ref[...], whh_ref[...]) + bg_ref[...]
        ).astype(jnp.bfloat16)
        i_g = jax.nn.sigmoid(gates[:, 0:HID])
        f_g = jax.nn.sigmoid(gates[:, HID : 2 * HID])
        g_g = jnp.tanh(gates[:, 2 * HID : 3 * HID])
        o_g = jax.nn.sigmoid(gates[:, 3 * HID : 4 * HID])
        c_new = (f_g * c_ref[...].astype(jnp.bfloat16) + i_g * g_g).astype(jnp.float32)
        h_new = o_g.astype(jnp.float32) * jnp.tanh(c_new)
        c_ref[...] = c_new
        h_ref[...] = h_new.astype(jnp.bfloat16)

    @pl.when(t == L // _T_PER_STEP - 1)
    def _finish():
        df = img_ref[...] + h_new
        out_ref[...] = jnp.maximum(_dot_t(df, wd1_ref[...]) + bd1_ref[...], 0.0)


def _lstm_fc(embd_tm, image, W_fc, b_fc, W_ih, W_hh, b_gates, W_d1, b_d1):
    return pl.pallas_call(
        _lstm_body,
        grid=(L // _T_PER_STEP,),
        in_specs=[
            pl.BlockSpec((_T_PER_STEP, B, EMB), lambda t: (t, 0, 0)),
            pl.BlockSpec((B, IMG_DIM), lambda t: (0, 0)),
            pl.BlockSpec((EMB, IMG_DIM), lambda t: (0, 0)),
            pl.BlockSpec((1, EMB), lambda t: (0, 0)),
            pl.BlockSpec((4 * HID, EMB), lambda t: (0, 0)),
            pl.BlockSpec((4 * HID, HID), lambda t: (0, 0)),
            pl.BlockSpec((1, 4 * HID), lambda t: (0, 0)),
            pl.BlockSpec((EMB, EMB), lambda t: (0, 0)),
            pl.BlockSpec((1, EMB), lambda t: (0, 0)),
        ],
        out_specs=pl.BlockSpec((B, EMB), lambda t: (0, 0)),
        out_shape=jax.ShapeDtypeStruct((B, EMB), jnp.float32),
        scratch_shapes=[
            pltpu.VMEM((B, HID), jnp.bfloat16),
            pltpu.VMEM((B, HID), jnp.float32),
            pltpu.VMEM((B, EMB), jnp.float32),
        ],
        compiler_params=pltpu.CompilerParams(
            dimension_semantics=("arbitrary",),
        ),
    )(embd_tm, image, W_fc, b_fc, W_ih, W_hh, b_gates, W_d1, b_d1)


_V_TILE = 2048
_N_VTILES = (VOCAB + _V_TILE - 1) // _V_TILE


def _proj_body(h1_ref, wd2_ref, bd2_ref, out_ref):
    out_ref[...] = _dot_t(h1_ref[...], wd2_ref[...]) + bd2_ref[...]


def _vocab_proj(h1, W_d2, b_d2):
    return pl.pallas_call(
        _proj_body,
        grid=(_N_VTILES,),
        in_specs=[
            pl.BlockSpec((B, EMB), lambda v: (0, 0)),
            pl.BlockSpec((_V_TILE, EMB), lambda v: (v, 0)),
            pl.BlockSpec((1, _V_TILE), lambda v: (0, v)),
        ],
        out_specs=pl.BlockSpec((B, _V_TILE), lambda v: (0, v)),
        out_shape=jax.ShapeDtypeStruct((B, VOCAB), jnp.float32),
        compiler_params=pltpu.CompilerParams(
            dimension_semantics=("arbitrary",),
        ),
    )(h1, W_d2, b_d2)


def kernel(image, caption, W_fc, b_fc, emb, W_ih, W_hh, b_ih, b_hh, W_d1, b_d1, W_d2, b_d2):
    idx_flat = caption.astype(jnp.int32).T.reshape(-1)
    embd = _sc_gather(emb, idx_flat)
    embd_tm = embd.reshape(L, B, EMB)
    h1 = _lstm_fc(
        embd_tm,
        image.astype(jnp.bfloat16),
        W_fc.astype(jnp.bfloat16),
        b_fc.reshape(1, EMB),
        W_ih.astype(jnp.bfloat16),
        W_hh.astype(jnp.bfloat16),
        (b_ih + b_hh).reshape(1, 4 * HID),
        W_d1.astype(jnp.bfloat16),
        b_d1.reshape(1, EMB),
    )
    return _vocab_proj(h1, W_d2, b_d2.reshape(1, VOCAB))
